# Initial kernel scaffold; baseline (speedup 1.0000x reference)
import jax
import jax.numpy as jnp
from jax.experimental import pallas as pl


def kernel(x, lengths, emb, W1, b1, W2, b2):
    raise NotImplementedError("write your pallas kernel here")


# speedup vs baseline: 13.3230x; 13.3230x over previous
import functools

import jax
import jax.numpy as jnp
from jax import lax
from jax.experimental import pallas as pl
from jax.experimental.pallas import tpu as pltpu
from jax.experimental.pallas import tpu_sc as plsc

_NC = 2
_NS = 16
_NW = _NC * _NS


@functools.lru_cache(maxsize=None)
def _embed_pool(B: int, L: int, V: int, D: int):
    BPW = B // _NW
    S0 = (min(L, 128) // 8) * 8
    S0 = min(S0, ((L + 1) // 2 + 7) // 8 * 8)
    S1 = L - S0
    assert 0 < S1 <= 128 and S0 % 8 == 0 and L % 8 == 0 and D % 16 == 0
    NDREG = D // 16
    mesh = plsc.VectorSubcoreMesh(core_axis_name="c", subcore_axis_name="s")

    @functools.partial(
        pl.kernel,
        mesh=mesh,
        compiler_params=pltpu.CompilerParams(use_tc_tiling_on_sc=False),
        out_type=jax.ShapeDtypeStruct((B, D), jnp.float32),
        scratch_types=[
            pltpu.VMEM((BPW, L), jnp.int32),
            pltpu.VMEM((2, L, D), jnp.float32),
            pltpu.VMEM((BPW, D), jnp.float32),
            pltpu.SemaphoreType.DMA,
            pltpu.SemaphoreType.DMA,
        ],
    )
    def ker(x_hbm, emb_hbm, out_hbm, idx_v, rows_v, rep_v, sem0, sem1):
        wid = lax.axis_index("s") * _NC + lax.axis_index("c")
        base = wid * BPW
        pltpu.sync_copy(x_hbm.at[pl.ds(base, BPW), :], idx_v)
        sems = (sem0, sem1)

        def issue(i, buf):
            sem = sems[buf]
            pltpu.async_copy(emb_hbm.at[idx_v.at[i, pl.ds(0, S0)]],
                             rows_v.at[buf, pl.ds(0, S0), :], sem)
            pltpu.async_copy(emb_hbm.at[idx_v.at[i, pl.ds(S0, S1)]],
                             rows_v.at[buf, pl.ds(S0, S1), :], sem)

        def drain(buf):
            pltpu.make_async_copy(emb_hbm.at[pl.ds(0, L), :],
                                  rows_v.at[buf], sems[buf]).wait()

        def accum(i, buf):
            def abody(j, accs):
                return tuple(a + rows_v[buf, j, pl.ds(16 * k, 16)]
                             for k, a in enumerate(accs))
            z = jnp.zeros((16,), jnp.float32)
            accs = lax.fori_loop(0, L, abody, (z,) * NDREG)
            for k in range(NDREG):
                rep_v[i, pl.ds(16 * k, 16)] = accs[k]

        issue(0, 0)

        def body(t, carry):
            i0 = 2 * t
            issue(i0 + 1, 1)
            drain(0)
            accum(i0, 0)

            @pl.when(i0 + 2 < BPW)
            def _():
                issue(i0 + 2, 0)

            drain(1)
            accum(i0 + 1, 1)
            return carry

        lax.fori_loop(0, BPW // 2, body, 0)
        pltpu.sync_copy(rep_v, out_hbm.at[pl.ds(base, BPW), :])

    return ker


@functools.lru_cache(maxsize=None)
def _mlp(B: int, D: int, H: int, C: int):
    def body(rep_ref, inv_ref, w1_ref, b1_ref, w2_ref, b2_ref, out_ref):
        r = rep_ref[...] * inv_ref[...]
        h = jnp.dot(r, w1_ref[...], preferred_element_type=jnp.float32)
        h = jnp.maximum(h + b1_ref[...], 0.0)
        o = jnp.dot(h, w2_ref[...], preferred_element_type=jnp.float32)
        out_ref[...] = o + b2_ref[...]

    return pl.pallas_call(
        body,
        out_shape=jax.ShapeDtypeStruct((B, C), jnp.float32),
    )


def kernel(x, lengths, emb, W1, b1, W2, b2):
    B, L = x.shape
    V, D = emb.shape
    H = W1.shape[1]
    C = W2.shape[1]
    rep = _embed_pool(B, L, V, D)(x, emb)
    inv_len = (1.0 / lengths.astype(jnp.float32)).reshape(B, 1)
    return _mlp(B, D, H, C)(rep, inv_len, W1, b1.reshape(1, H),
                            W2, b2.reshape(1, C))


# speedup vs baseline: 17.3673x; 1.3036x over previous
import functools

import jax
import jax.numpy as jnp
from jax import lax
from jax.experimental import pallas as pl
from jax.experimental.pallas import tpu as pltpu
from jax.experimental.pallas import tpu_sc as plsc

_NC = 2
_NS = 16
_NW = _NC * _NS


@functools.lru_cache(maxsize=None)
def _embed_pool(B: int, L: int, V: int, D: int):
    BPW = B // _NW
    S0 = (min(L, 128) // 8) * 8
    S0 = min(S0, ((L + 1) // 2 + 7) // 8 * 8)
    S1 = L - S0
    assert 0 < S1 <= 128 and S0 % 8 == 0 and L % 8 == 0 and D % 16 == 0
    NDREG = D // 16
    mesh = plsc.VectorSubcoreMesh(core_axis_name="c", subcore_axis_name="s")

    NBUF = 4
    UNROLL = 8
    assert BPW % NBUF == 0 and L % UNROLL == 0

    @functools.partial(
        pl.kernel,
        mesh=mesh,
        compiler_params=pltpu.CompilerParams(use_tc_tiling_on_sc=False),
        out_type=jax.ShapeDtypeStruct((B, D), jnp.float32),
        scratch_types=[
            pltpu.VMEM((BPW, L), jnp.int32),
            pltpu.VMEM((NBUF, L, D), jnp.float32),
            pltpu.VMEM((BPW, D), jnp.float32),
            [pltpu.SemaphoreType.DMA] * NBUF,
        ],
    )
    def ker(x_hbm, emb_hbm, out_hbm, idx_v, rows_v, rep_v, sems):
        wid = lax.axis_index("s") * _NC + lax.axis_index("c")
        base = wid * BPW
        pltpu.sync_copy(x_hbm.at[pl.ds(base, BPW), :], idx_v)

        def issue(i, buf):
            sem = sems[buf]
            pltpu.async_copy(emb_hbm.at[idx_v.at[i, pl.ds(0, S0)]],
                             rows_v.at[buf, pl.ds(0, S0), :], sem)
            pltpu.async_copy(emb_hbm.at[idx_v.at[i, pl.ds(S0, S1)]],
                             rows_v.at[buf, pl.ds(S0, S1), :], sem)

        def drain(buf):
            pltpu.make_async_copy(emb_hbm.at[pl.ds(0, L), :],
                                  rows_v.at[buf], sems[buf]).wait()

        def accum(i, buf):
            def abody(j, accs):
                accs = list(accs)
                for u in range(UNROLL):
                    for k in range(NDREG):
                        accs[u % 2 * NDREG + k] += (
                            rows_v[buf, j * UNROLL + u, pl.ds(16 * k, 16)])
                return tuple(accs)
            z = jnp.zeros((16,), jnp.float32)
            accs = lax.fori_loop(0, L // UNROLL, abody, (z,) * (2 * NDREG))
            for k in range(NDREG):
                rep_v[i, pl.ds(16 * k, 16)] = accs[k] + accs[NDREG + k]

        for u in range(NBUF):
            issue(u, u)

        def body(t, carry):
            i0 = NBUF * t
            for u in range(NBUF):
                drain(u)
                accum(i0 + u, u)
                issue(i0 + u + NBUF, u)
            return carry

        lax.fori_loop(0, BPW // NBUF - 1, body, 0)
        i0 = BPW - NBUF
        for u in range(NBUF):
            drain(u)
            accum(i0 + u, u)
        pltpu.sync_copy(rep_v, out_hbm.at[pl.ds(base, BPW), :])

    return ker


@functools.lru_cache(maxsize=None)
def _mlp(B: int, D: int, H: int, C: int):
    def body(rep_ref, inv_ref, w1_ref, b1_ref, w2_ref, b2_ref, out_ref):
        r = rep_ref[...] * inv_ref[...]
        h = jnp.dot(r, w1_ref[...], preferred_element_type=jnp.float32)
        h = jnp.maximum(h + b1_ref[...], 0.0)
        o = jnp.dot(h, w2_ref[...], preferred_element_type=jnp.float32)
        out_ref[...] = o + b2_ref[...]

    return pl.pallas_call(
        body,
        out_shape=jax.ShapeDtypeStruct((B, C), jnp.float32),
    )


def kernel(x, lengths, emb, W1, b1, W2, b2):
    B, L = x.shape
    V, D = emb.shape
    H = W1.shape[1]
    C = W2.shape[1]
    rep = _embed_pool(B, L, V, D)(x, emb)
    inv_len = (1.0 / lengths.astype(jnp.float32)).reshape(B, 1)
    return _mlp(B, D, H, C)(rep, inv_len, W1, b1.reshape(1, H),
                            W2, b2.reshape(1, C))
